# Initial kernel scaffold; baseline (speedup 1.0000x reference)
import jax
import jax.numpy as jnp
from jax.experimental import pallas as pl


def kernel(x, edge_index, W1, b1, W2, b2, F, emb):
    raise NotImplementedError("write your pallas kernel here")


# speedup vs baseline: 6.4079x; 6.4079x over previous
import functools
import jax
import jax.numpy as jnp
from jax import lax
from jax.experimental import pallas as pl
from jax.experimental.pallas import tpu as pltpu
from jax.experimental.pallas import tpu_sc as plsc

N = 10000
E = 320000
CH = 128
KAPPA = 0.95

NC, NS = 2, 16
NW = NC * NS
EPW = E // NW
C = 80
NCHUNK = EPW // C
NP = 10240
RPS = NP // NS

BR = 1000


def _tc1_body(x_ref, emb_ref, W1_ref, F_ref, u1_ref, e2_ref, wp_ref):
    @pl.when(pl.program_id(0) == 0)
    def _():
        Fm = F_ref[...]
        Wp = lax.dot_general(Fm, Fm, (((0,), (0,)), ((), ())),
                             preferred_element_type=jnp.float32)
        nrm = jnp.sqrt(jnp.sum(Wp * Wp))
        Wp = jnp.where(nrm > 1.0, Wp / (nrm + 1e-5), Wp)
        wp_ref[...] = Wp * KAPPA
    u1_ref[...] = jnp.dot(x_ref[...], W1_ref[...],
                          preferred_element_type=jnp.float32)
    e2_ref[...] = jnp.dot(emb_ref[...], wp_ref[...],
                          preferred_element_type=jnp.float32)


_tc1 = pl.pallas_call(
    _tc1_body,
    grid=(N // BR,),
    in_specs=[
        pl.BlockSpec((BR, CH), lambda i: (i, 0)),
        pl.BlockSpec((BR, CH), lambda i: (i, 0)),
        pl.BlockSpec((CH, CH), lambda i: (0, 0)),
        pl.BlockSpec((CH, CH), lambda i: (0, 0)),
    ],
    out_specs=[pl.BlockSpec((BR, CH), lambda i: (i, 0))] * 2,
    out_shape=[jax.ShapeDtypeStruct((N, CH), jnp.float32)] * 2,
    scratch_shapes=[pltpu.VMEM((CH, CH), jnp.float32)],
)


def _tc2_body(p0_ref, p1_ref, b1_ref, W2_ref, e2_ref, u2_ref):
    h = jnp.maximum(p0_ref[...] + p1_ref[...] + b1_ref[...], 0.0)
    u2_ref[...] = jnp.dot(h, W2_ref[...],
                          preferred_element_type=jnp.float32) + e2_ref[...]


_tc2 = pl.pallas_call(
    _tc2_body,
    grid=(N // BR,),
    in_specs=[
        pl.BlockSpec((BR, CH), lambda i: (i, 0)),
        pl.BlockSpec((BR, CH), lambda i: (i, 0)),
        pl.BlockSpec((1, CH), lambda i: (0, 0)),
        pl.BlockSpec((CH, CH), lambda i: (0, 0)),
        pl.BlockSpec((BR, CH), lambda i: (i, 0)),
    ],
    out_specs=pl.BlockSpec((BR, CH), lambda i: (i, 0)),
    out_shape=jax.ShapeDtypeStruct((N, CH), jnp.float32),
)


def _tc3_body(q0_ref, q1_ref, b2_ref, out_ref):
    out_ref[...] = q0_ref[...] + q1_ref[...] + b2_ref[...]


_tc3 = pl.pallas_call(
    _tc3_body,
    grid=(N // BR,),
    in_specs=[
        pl.BlockSpec((BR, CH), lambda i: (i, 0)),
        pl.BlockSpec((BR, CH), lambda i: (i, 0)),
        pl.BlockSpec((1, CH), lambda i: (0, 0)),
    ],
    out_specs=pl.BlockSpec((BR, CH), lambda i: (i, 0)),
    out_shape=jax.ShapeDtypeStruct((N, CH), jnp.float32),
)


_sc_mesh = plsc.VectorSubcoreMesh(
    core_axis_name="c", subcore_axis_name="s", num_cores=NC, num_subcores=NS)


@functools.partial(
    pl.kernel,
    out_type=jax.ShapeDtypeStruct((NC, NP, CH), jnp.float32),
    mesh=_sc_mesh,
    scratch_types=[
        pltpu.VMEM((NCHUNK, C), jnp.int32),
        pltpu.VMEM((NCHUNK, C), jnp.int32),
        pltpu.VMEM((C, CH), jnp.float32),
        pltpu.VMEM_SHARED((NP, CH), jnp.float32),
        pltpu.SemaphoreType.DMA,
    ],
)
def _segsum(u_hbm, src_hbm, dst_hbm, zeros_hbm, out_hbm,
            src_v, dst_v, rows_v, acc, sem):
    cid = lax.axis_index("c")
    sid = lax.axis_index("s")
    wid = sid * NC + cid
    pltpu.sync_copy(zeros_hbm.at[pl.ds(sid * RPS, RPS)],
                    acc.at[pl.ds(sid * RPS, RPS)])
    pltpu.sync_copy(src_hbm.at[wid], src_v)
    pltpu.sync_copy(dst_hbm.at[wid], dst_v)
    plsc.subcore_barrier()

    def body(j, carry):
        pltpu.async_copy(u_hbm.at[src_v.at[j]], rows_v, sem).wait()
        pltpu.sync_copy(rows_v, acc.at[dst_v.at[j]], add=True)
        return carry

    lax.fori_loop(0, NCHUNK, body, 0)
    plsc.subcore_barrier()
    pltpu.sync_copy(acc.at[pl.ds(sid * RPS, RPS)],
                    out_hbm.at[cid, pl.ds(sid * RPS, RPS)])


def kernel(x, edge_index, W1, b1, W2, b2, F, emb):
    src = edge_index[0].reshape(NW, NCHUNK, C)
    dst = edge_index[1].reshape(NW, NCHUNK, C)
    zeros = jnp.zeros((NP, CH), jnp.float32)
    u1, e2 = _tc1(x, emb, W1, F)
    p = _segsum(u1, src, dst, zeros)
    u2 = _tc2(p[0, :N], p[1, :N], b1.reshape(1, CH), W2, e2)
    q = _segsum(u2, src, dst, zeros)
    return _tc3(q[0, :N], q[1, :N], b2.reshape(1, CH))
